# baseline (speedup 1.0000x reference)
import functools

import jax
import jax.numpy as jnp
from jax import lax
from jax.experimental import pallas as pl
from jax.experimental.pallas import tpu as pltpu
from jax.experimental.pallas import tpu_sc as plsc

THRESH = 0.7
N_MIN = 131072
IGNORE = 255

N, C, H, W = 8, 19, 512, 512
NPIX = N * H * W


_BH = 64


def _nll_pick_body(logits_ref, labels_ref, key_ref, nll_ref):
    lb = labels_ref[0]
    invalid = lb == IGNORE
    lb0 = jnp.where(invalid, 0, lb)

    x0 = logits_ref[0, 0]
    m = x0
    for c in range(1, C):
        m = jnp.maximum(m, logits_ref[0, c])

    s = jnp.zeros_like(m)
    xl = jnp.zeros_like(m)
    for c in range(C):
        xc = logits_ref[0, c]
        s = s + jnp.exp(xc - m)
        xl = xl + jnp.where(lb0 == c, xc, 0.0)

    pick = jnp.exp(xl - m) / s
    pick = jnp.where(invalid, 1.0, pick)
    nll = m + jnp.log(s) - xl
    key_ref[0] = lax.bitcast_convert_type(pick, jnp.int32)
    nll_ref[0] = nll


def _nll_pick(logits, labels):
    grid = (N, H // _BH)
    return pl.pallas_call(
        _nll_pick_body,
        grid=grid,
        in_specs=[
            pl.BlockSpec((1, C, _BH, W), lambda n, h: (n, 0, h, 0)),
            pl.BlockSpec((1, _BH, W), lambda n, h: (n, h, 0)),
        ],
        out_specs=[
            pl.BlockSpec((1, _BH, W), lambda n, h: (n, h, 0)),
            pl.BlockSpec((1, _BH, W), lambda n, h: (n, h, 0)),
        ],
        out_shape=[
            jax.ShapeDtypeStruct((N, H, W), jnp.int32),
            jax.ShapeDtypeStruct((N, H, W), jnp.float32),
        ],
    )(logits, labels)


_NB = 2048
_NBP = _NB + 1
_TROWS = H // 2
_CROWS = 64
_NCHUNK = _TROWS // _CROWS


def _lanes():
    return lax.iota(jnp.int32, 16)


def _shr(key, amount):
    return lax.shift_right_logical(key, jnp.full((16,), amount, jnp.int32))


def _zero_hist(hist):
    z = jnp.zeros((16,), jnp.int32)

    def body(i, _):
        def inner(l, _):
            hist[l, pl.ds(i * 16, 16)] = z
            return 0
        lax.fori_loop(0, 16, inner, 0, unroll=16)
        return 0

    lax.fori_loop(0, _NB // 16, body, 0)


def _level(keys_hbm, buf, hist, mbuf, total, shared, sid, img, half, rank,
           bin_fn, mask_fn):
    lanes = _lanes()
    ones = jnp.full((16,), 1, jnp.int32)

    _zero_hist(hist)

    def chunk_body(ch, _):
        pltpu.sync_copy(
            keys_hbm.at[img, pl.ds(half * _TROWS + ch * _CROWS, _CROWS), :],
            buf)

        def row_body(r, _):
            def vec_body(c, _):
                key = buf[r, pl.ds(c * 16, 16)]
                plsc.addupdate_scatter(hist, [lanes, bin_fn(key)], ones,
                                       mask=mask_fn(key))
                return 0

            lax.fori_loop(0, W // 16, vec_body, 0, unroll=8)
            return 0

        lax.fori_loop(0, _CROWS, row_body, 0)
        return 0

    lax.fori_loop(0, _NCHUNK, chunk_body, 0)

    def red_body(i, _):
        acc = hist[0, pl.ds(i * 16, 16)]
        for l in range(1, 16):
            acc = acc + hist[l, pl.ds(i * 16, 16)]
        total[pl.ds(i * 16, 16)] = acc
        return 0

    lax.fori_loop(0, _NB // 16, red_body, 0)

    pltpu.sync_copy(total, shared.at[sid])
    plsc.subcore_barrier()
    pltpu.sync_copy(shared, mbuf)
    plsc.subcore_barrier()

    def scan_body(i, carry):
        prefix, bcnt, cbef = carry
        v = mbuf[0, pl.ds(i * 16, 16)]
        for r in range(1, 16):
            v = v + mbuf[r, pl.ds(i * 16, 16)]
        cs = plsc.cumsum(v) + prefix
        le = cs <= rank
        bcnt = bcnt + jnp.sum(jnp.where(le, 1, 0))
        cbef = cbef + jnp.sum(jnp.where(le, v, 0))
        prefix = jnp.max(cs)
        return prefix, bcnt, cbef

    z = jnp.int32(0)
    _, b, cbefore = lax.fori_loop(0, _NB // 16, scan_body, (z, z, z))
    return b, rank - cbefore


def _true_mask(key):
    return jnp.full((16,), True)


def _select_body(keys_hbm, kth_hbm, buf, hist, mbuf, total, kthbuf, shared):
    cid = lax.axis_index("c")
    sid = lax.axis_index("s")
    img = sid >> 1
    half = sid & 1

    b1, r1 = _level(keys_hbm, buf, hist, mbuf, total, shared, sid, img, half,
                    jnp.int32(N_MIN), lambda key: _shr(key, 21), _true_mask)

    m1 = jnp.broadcast_to(b1, (16,)).astype(jnp.int32)
    b2, r2 = _level(
        keys_hbm, buf, hist, mbuf, total, shared, sid, img, half, r1,
        lambda key: jnp.bitwise_and(_shr(key, 10),
                                    jnp.full((16,), 0x7FF, jnp.int32)),
        lambda key: _shr(key, 21) == m1)

    p2 = b1 * 2048 + b2
    m2 = jnp.broadcast_to(p2, (16,)).astype(jnp.int32)
    b3, _r3 = _level(
        keys_hbm, buf, hist, mbuf, total, shared, sid, img, half, r2,
        lambda key: jnp.bitwise_and(key, jnp.full((16,), 0x3FF, jnp.int32)),
        lambda key: _shr(key, 10) == m2)

    @pl.when(jnp.logical_and(cid == 0, sid == 0))
    def _():
        kthbuf[...] = jnp.broadcast_to(p2 * 1024 + b3, (16,)).astype(jnp.int32)
        pltpu.sync_copy(kthbuf, kth_hbm)


@functools.cache
def _sc_select():
    mesh = plsc.VectorSubcoreMesh(core_axis_name="c", subcore_axis_name="s")
    cp = pltpu.CompilerParams(needs_layout_passes=False)
    i32 = jnp.int32
    return pl.kernel(
        _select_body, mesh=mesh, compiler_params=cp,
        out_type=jax.ShapeDtypeStruct((16,), i32),
        scratch_types=[
            pltpu.VMEM((_CROWS, W), i32),
            pltpu.VMEM((16, _NBP), i32),
            pltpu.VMEM((16, _NB), i32),
            pltpu.VMEM((_NB,), i32),
            pltpu.VMEM((16,), i32),
            pltpu.VMEM_SHARED((16, _NB), i32),
        ],
    )


_KEY_07 = 0x3F333333


def _loss_body(kth_ref, key_ref, nll_ref, out_ref, acc_ref):
    step = pl.program_id(0)

    @pl.when(step == 0)
    def _():
        acc_ref[0] = 0.0
        acc_ref[1] = 0.0

    thresh_key = jnp.maximum(kth_ref[0, 0], _KEY_07)
    k = key_ref[0]
    nl = nll_ref[0]
    keep = k <= thresh_key
    acc_ref[0] += jnp.sum(jnp.where(keep, nl, 0.0))
    acc_ref[1] += jnp.sum(keep.astype(jnp.float32))

    @pl.when(step == pl.num_programs(0) - 1)
    def _():
        out_ref[0, 0] = acc_ref[0] / jnp.maximum(acc_ref[1], 1.0)


def _masked_ce(kth, keys, nll):
    return pl.pallas_call(
        _loss_body,
        grid=(N,),
        in_specs=[
            pl.BlockSpec(memory_space=pltpu.SMEM),
            pl.BlockSpec((1, H, W), lambda i: (i, 0, 0)),
            pl.BlockSpec((1, H, W), lambda i: (i, 0, 0)),
        ],
        out_specs=pl.BlockSpec(memory_space=pltpu.SMEM),
        out_shape=jax.ShapeDtypeStruct((1, 1), jnp.float32),
        scratch_shapes=[pltpu.SMEM((2,), jnp.float32)],
    )(kth, keys, nll)


def kernel(logits, labels):
    keys, nll = _nll_pick(logits, labels)
    kth = _sc_select()(keys)
    loss = _masked_ce(kth.reshape(1, 16), keys, nll)
    return loss.reshape(())


# speedup vs baseline: 1.8933x; 1.8933x over previous
import functools

import jax
import jax.numpy as jnp
from jax import lax
from jax.experimental import pallas as pl
from jax.experimental.pallas import tpu as pltpu
from jax.experimental.pallas import tpu_sc as plsc

THRESH = 0.7
N_MIN = 131072
IGNORE = 255

N, C, H, W = 8, 19, 512, 512
NPIX = N * H * W


_BH = 64


def _nll_pick_body(logits_ref, labels_ref, key_ref, nll_ref):
    lb = labels_ref[0]
    invalid = lb == IGNORE
    lb0 = jnp.where(invalid, 0, lb)

    x0 = logits_ref[0, 0]
    m = x0
    for c in range(1, C):
        m = jnp.maximum(m, logits_ref[0, c])

    s = jnp.zeros_like(m)
    xl = jnp.zeros_like(m)
    for c in range(C):
        xc = logits_ref[0, c]
        s = s + jnp.exp(xc - m)
        xl = xl + jnp.where(lb0 == c, xc, 0.0)

    pick = jnp.exp(xl - m) / s
    pick = jnp.where(invalid, 1.0, pick)
    nll = m + jnp.log(s) - xl
    key_ref[0] = lax.bitcast_convert_type(pick, jnp.int32)
    nll_ref[0] = nll


def _nll_pick(logits, labels):
    grid = (N, H // _BH)
    return pl.pallas_call(
        _nll_pick_body,
        grid=grid,
        in_specs=[
            pl.BlockSpec((1, C, _BH, W), lambda n, h: (n, 0, h, 0)),
            pl.BlockSpec((1, _BH, W), lambda n, h: (n, h, 0)),
        ],
        out_specs=[
            pl.BlockSpec((1, _BH, W), lambda n, h: (n, h, 0)),
            pl.BlockSpec((1, _BH, W), lambda n, h: (n, h, 0)),
        ],
        out_shape=[
            jax.ShapeDtypeStruct((N, H, W), jnp.int32),
            jax.ShapeDtypeStruct((N, H, W), jnp.float32),
        ],
    )(logits, labels)


_NB = 2048
_NBP = _NB + 1
_TROWS = H // 2
_CROWS = 64
_NCHUNK = _TROWS // _CROWS


def _lanes():
    return lax.iota(jnp.int32, 16)


def _shr(key, amount):
    return lax.shift_right_logical(key, jnp.full((16,), amount, jnp.int32))


def _zero_hist(hist):
    z = jnp.zeros((16,), jnp.int32)

    @plsc.parallel_loop(0, 16 * (_NB // 16), unroll=8)
    def _(i):
        hist[i >> 7, pl.ds((i & 127) * 16, 16)] = z


def _level(keys_hbm, buf, hist, mbuf, total, shared, sid, img, half, rank,
           bin_fn, mask_fn):
    lanes = _lanes()
    ones = jnp.full((16,), 1, jnp.int32)

    _zero_hist(hist)

    def chunk_body(ch, _):
        pltpu.sync_copy(
            keys_hbm.at[img, pl.ds(half * _TROWS + ch * _CROWS, _CROWS), :],
            buf)

        @plsc.parallel_loop(0, _CROWS * (W // 16), unroll=8)
        def _(i):
            key = buf[i >> 5, pl.ds((i & 31) * 16, 16)]
            plsc.addupdate_scatter(hist, [lanes, bin_fn(key)], ones,
                                   mask=mask_fn(key))

        return 0

    lax.fori_loop(0, _NCHUNK, chunk_body, 0)

    @plsc.parallel_loop(0, _NB // 16, unroll=2)
    def _(i):
        acc = hist[0, pl.ds(i * 16, 16)]
        for l in range(1, 16):
            acc = acc + hist[l, pl.ds(i * 16, 16)]
        total[pl.ds(i * 16, 16)] = acc

    pltpu.sync_copy(total, shared.at[sid])
    plsc.subcore_barrier()
    pltpu.sync_copy(shared, mbuf)
    plsc.subcore_barrier()

    def scan_body(i, carry):
        prefix, bcnt, cbef = carry
        v = mbuf[0, pl.ds(i * 16, 16)]
        for r in range(1, 16):
            v = v + mbuf[r, pl.ds(i * 16, 16)]
        cs = plsc.cumsum(v) + prefix
        le = cs <= rank
        bcnt = bcnt + jnp.sum(jnp.where(le, 1, 0))
        cbef = cbef + jnp.sum(jnp.where(le, v, 0))
        prefix = jnp.max(cs)
        return prefix, bcnt, cbef

    z = jnp.int32(0)
    _, b, cbefore = lax.fori_loop(0, _NB // 16, scan_body, (z, z, z))
    return b, rank - cbefore


def _true_mask(key):
    return jnp.full((16,), True)


def _select_body(keys_hbm, kth_hbm, buf, hist, mbuf, total, kthbuf, shared):
    cid = lax.axis_index("c")
    sid = lax.axis_index("s")
    img = sid >> 1
    half = sid & 1

    b1, r1 = _level(keys_hbm, buf, hist, mbuf, total, shared, sid, img, half,
                    jnp.int32(N_MIN), lambda key: _shr(key, 21), _true_mask)

    m1 = jnp.broadcast_to(b1, (16,)).astype(jnp.int32)
    b2, r2 = _level(
        keys_hbm, buf, hist, mbuf, total, shared, sid, img, half, r1,
        lambda key: jnp.bitwise_and(_shr(key, 10),
                                    jnp.full((16,), 0x7FF, jnp.int32)),
        lambda key: _shr(key, 21) == m1)

    p2 = b1 * 2048 + b2
    m2 = jnp.broadcast_to(p2, (16,)).astype(jnp.int32)
    b3, _r3 = _level(
        keys_hbm, buf, hist, mbuf, total, shared, sid, img, half, r2,
        lambda key: jnp.bitwise_and(key, jnp.full((16,), 0x3FF, jnp.int32)),
        lambda key: _shr(key, 10) == m2)

    @pl.when(jnp.logical_and(cid == 0, sid == 0))
    def _():
        kthbuf[...] = jnp.broadcast_to(p2 * 1024 + b3, (16,)).astype(jnp.int32)
        pltpu.sync_copy(kthbuf, kth_hbm)


@functools.cache
def _sc_select():
    mesh = plsc.VectorSubcoreMesh(core_axis_name="c", subcore_axis_name="s")
    cp = pltpu.CompilerParams(needs_layout_passes=False)
    i32 = jnp.int32
    return pl.kernel(
        _select_body, mesh=mesh, compiler_params=cp,
        out_type=jax.ShapeDtypeStruct((16,), i32),
        scratch_types=[
            pltpu.VMEM((_CROWS, W), i32),
            pltpu.VMEM((16, _NBP), i32),
            pltpu.VMEM((16, _NB), i32),
            pltpu.VMEM((_NB,), i32),
            pltpu.VMEM((16,), i32),
            pltpu.VMEM_SHARED((16, _NB), i32),
        ],
    )


_KEY_07 = 0x3F333333


def _loss_body(kth_ref, key_ref, nll_ref, out_ref, acc_ref):
    step = pl.program_id(0)

    @pl.when(step == 0)
    def _():
        acc_ref[0] = 0.0
        acc_ref[1] = 0.0

    thresh_key = jnp.maximum(kth_ref[0, 0], _KEY_07)
    k = key_ref[0]
    nl = nll_ref[0]
    keep = k <= thresh_key
    acc_ref[0] += jnp.sum(jnp.where(keep, nl, 0.0))
    acc_ref[1] += jnp.sum(keep.astype(jnp.float32))

    @pl.when(step == pl.num_programs(0) - 1)
    def _():
        out_ref[0, 0] = acc_ref[0] / jnp.maximum(acc_ref[1], 1.0)


def _masked_ce(kth, keys, nll):
    return pl.pallas_call(
        _loss_body,
        grid=(N,),
        in_specs=[
            pl.BlockSpec(memory_space=pltpu.SMEM),
            pl.BlockSpec((1, H, W), lambda i: (i, 0, 0)),
            pl.BlockSpec((1, H, W), lambda i: (i, 0, 0)),
        ],
        out_specs=pl.BlockSpec(memory_space=pltpu.SMEM),
        out_shape=jax.ShapeDtypeStruct((1, 1), jnp.float32),
        scratch_shapes=[pltpu.SMEM((2,), jnp.float32)],
    )(kth, keys, nll)


def kernel(logits, labels):
    keys, nll = _nll_pick(logits, labels)
    kth = _sc_select()(keys)
    loss = _masked_ce(kth.reshape(1, 16), keys, nll)
    return loss.reshape(())


# speedup vs baseline: 1.9232x; 1.0158x over previous
import functools

import jax
import jax.numpy as jnp
from jax import lax
from jax.experimental import pallas as pl
from jax.experimental.pallas import tpu as pltpu
from jax.experimental.pallas import tpu_sc as plsc

THRESH = 0.7
N_MIN = 131072
IGNORE = 255

N, C, H, W = 8, 19, 512, 512
NPIX = N * H * W


_BH = 64


def _nll_pick_body(logits_ref, labels_ref, key_ref, nll_ref):
    lb = labels_ref[0]
    invalid = lb == IGNORE
    lb0 = jnp.where(invalid, 0, lb)

    x0 = logits_ref[0, 0]
    m = x0
    for c in range(1, C):
        m = jnp.maximum(m, logits_ref[0, c])

    s = jnp.zeros_like(m)
    xl = jnp.zeros_like(m)
    for c in range(C):
        xc = logits_ref[0, c]
        s = s + jnp.exp(xc - m)
        xl = xl + jnp.where(lb0 == c, xc, 0.0)

    pick = jnp.exp(xl - m) / s
    pick = jnp.where(invalid, 1.0, pick)
    nll = m + jnp.log(s) - xl
    key_ref[0] = lax.bitcast_convert_type(pick, jnp.int32)
    nll_ref[0] = nll


def _nll_pick(logits, labels):
    grid = (N, H // _BH)
    return pl.pallas_call(
        _nll_pick_body,
        grid=grid,
        in_specs=[
            pl.BlockSpec((1, C, _BH, W), lambda n, h: (n, 0, h, 0)),
            pl.BlockSpec((1, _BH, W), lambda n, h: (n, h, 0)),
        ],
        out_specs=[
            pl.BlockSpec((1, _BH, W), lambda n, h: (n, h, 0)),
            pl.BlockSpec((1, _BH, W), lambda n, h: (n, h, 0)),
        ],
        out_shape=[
            jax.ShapeDtypeStruct((N, H, W), jnp.int32),
            jax.ShapeDtypeStruct((N, H, W), jnp.float32),
        ],
    )(logits, labels)


_NB = 2048
_NBP = _NB + 1
_TROWS = H // 2
_CROWS = 64
_NCHUNK = _TROWS // _CROWS


def _lanes():
    return lax.iota(jnp.int32, 16)


def _shr(key, amount):
    return lax.shift_right_logical(key, jnp.full((16,), amount, jnp.int32))


def _zero_hist(hist):
    z = jnp.zeros((16,), jnp.int32)

    @plsc.parallel_loop(0, 16 * (_NB // 16), unroll=8)
    def _(i):
        hist[i >> 7, pl.ds((i & 127) * 16, 16)] = z


def _level(keys_hbm, buf, hist, mbuf, total, shared, sid, img, half, rank,
           bin_fn, mask_fn):
    lanes = _lanes()
    ones = jnp.full((16,), 1, jnp.int32)

    _zero_hist(hist)

    def chunk_body(ch, _):
        pltpu.sync_copy(
            keys_hbm.at[img, pl.ds(half * _TROWS + ch * _CROWS, _CROWS), :],
            buf)

        @plsc.parallel_loop(0, _CROWS * (W // 16), unroll=8)
        def _(i):
            key = buf[i >> 5, pl.ds((i & 31) * 16, 16)]
            plsc.addupdate_scatter(hist, [lanes, bin_fn(key)], ones,
                                   mask=mask_fn(key))

        return 0

    lax.fori_loop(0, _NCHUNK, chunk_body, 0)

    @plsc.parallel_loop(0, _NB // 16, unroll=2)
    def _(i):
        acc = hist[0, pl.ds(i * 16, 16)]
        for l in range(1, 16):
            acc = acc + hist[l, pl.ds(i * 16, 16)]
        total[pl.ds(i * 16, 16)] = acc

    pltpu.sync_copy(total, shared.at[sid])
    plsc.subcore_barrier()
    pltpu.sync_copy(shared, mbuf)
    plsc.subcore_barrier()

    def scan_body(i, carry):
        prefix, bcnt, cbef = carry
        v = mbuf[0, pl.ds(i * 16, 16)]
        for r in range(1, 16):
            v = v + mbuf[r, pl.ds(i * 16, 16)]
        cs = plsc.cumsum(v) + prefix
        le = cs <= rank
        bcnt = bcnt + jnp.sum(jnp.where(le, 1, 0))
        cbef = cbef + jnp.sum(jnp.where(le, v, 0))
        prefix = jnp.max(cs)
        return prefix, bcnt, cbef

    z = jnp.int32(0)
    _, b, cbefore = lax.fori_loop(0, _NB // 16, scan_body, (z, z, z))
    return b, rank - cbefore


def _true_mask(key):
    return jnp.full((16,), True)


def _select_body(keys_hbm, kth_hbm, buf, hist, mbuf, total, kthbuf, shared):
    cid = lax.axis_index("c")
    sid = lax.axis_index("s")
    img = sid >> 1
    half = sid & 1

    @pl.when(cid == 0)
    def _():
        b1, r1 = _level(keys_hbm, buf, hist, mbuf, total, shared, sid, img,
                        half, jnp.int32(N_MIN), lambda key: _shr(key, 21),
                        _true_mask)

        m1 = jnp.broadcast_to(b1, (16,)).astype(jnp.int32)
        b2, r2 = _level(
            keys_hbm, buf, hist, mbuf, total, shared, sid, img, half, r1,
            lambda key: jnp.bitwise_and(_shr(key, 10),
                                        jnp.full((16,), 0x7FF, jnp.int32)),
            lambda key: _shr(key, 21) == m1)

        p2 = b1 * 2048 + b2
        m2 = jnp.broadcast_to(p2, (16,)).astype(jnp.int32)
        b3, _r3 = _level(
            keys_hbm, buf, hist, mbuf, total, shared, sid, img, half, r2,
            lambda key: jnp.bitwise_and(key,
                                        jnp.full((16,), 0x3FF, jnp.int32)),
            lambda key: _shr(key, 10) == m2)

        @pl.when(sid == 0)
        def _():
            kthbuf[...] = jnp.broadcast_to(p2 * 1024 + b3,
                                           (16,)).astype(jnp.int32)
            pltpu.sync_copy(kthbuf, kth_hbm)


@functools.cache
def _sc_select():
    mesh = plsc.VectorSubcoreMesh(core_axis_name="c", subcore_axis_name="s")
    cp = pltpu.CompilerParams(needs_layout_passes=False)
    i32 = jnp.int32
    return pl.kernel(
        _select_body, mesh=mesh, compiler_params=cp,
        out_type=jax.ShapeDtypeStruct((16,), i32),
        scratch_types=[
            pltpu.VMEM((_CROWS, W), i32),
            pltpu.VMEM((16, _NBP), i32),
            pltpu.VMEM((16, _NB), i32),
            pltpu.VMEM((_NB,), i32),
            pltpu.VMEM((16,), i32),
            pltpu.VMEM_SHARED((16, _NB), i32),
        ],
    )


_KEY_07 = 0x3F333333


def _loss_body(kth_ref, key_ref, nll_ref, out_ref, acc_ref):
    step = pl.program_id(0)

    @pl.when(step == 0)
    def _():
        acc_ref[0] = 0.0
        acc_ref[1] = 0.0

    thresh_key = jnp.maximum(kth_ref[0, 0], _KEY_07)
    k = key_ref[0]
    nl = nll_ref[0]
    keep = k <= thresh_key
    acc_ref[0] += jnp.sum(jnp.where(keep, nl, 0.0))
    acc_ref[1] += jnp.sum(keep.astype(jnp.float32))

    @pl.when(step == pl.num_programs(0) - 1)
    def _():
        out_ref[0, 0] = acc_ref[0] / jnp.maximum(acc_ref[1], 1.0)


def _masked_ce(kth, keys, nll):
    return pl.pallas_call(
        _loss_body,
        grid=(N,),
        in_specs=[
            pl.BlockSpec(memory_space=pltpu.SMEM),
            pl.BlockSpec((1, H, W), lambda i: (i, 0, 0)),
            pl.BlockSpec((1, H, W), lambda i: (i, 0, 0)),
        ],
        out_specs=pl.BlockSpec(memory_space=pltpu.SMEM),
        out_shape=jax.ShapeDtypeStruct((1, 1), jnp.float32),
        scratch_shapes=[pltpu.SMEM((2,), jnp.float32)],
    )(kth, keys, nll)


def kernel(logits, labels):
    keys, nll = _nll_pick(logits, labels)
    kth = _sc_select()(keys)
    loss = _masked_ce(kth.reshape(1, 16), keys, nll)
    return loss.reshape(())


# speedup vs baseline: 2.9680x; 1.5433x over previous
import functools

import jax
import jax.numpy as jnp
from jax import lax
from jax.experimental import pallas as pl
from jax.experimental.pallas import tpu as pltpu
from jax.experimental.pallas import tpu_sc as plsc

THRESH = 0.7
N_MIN = 131072
IGNORE = 255

N, C, H, W = 8, 19, 512, 512
NPIX = N * H * W


_BH = 64


_KEY_07 = 0x3F333333


def _nll_pick_body(logits_ref, labels_ref, key_ref, nll_ref, cnt_ref,
                   acc_ref):
    step = pl.program_id(0) * pl.num_programs(1) + pl.program_id(1)

    @pl.when(step == 0)
    def _():
        acc_ref[0] = 0

    lb = labels_ref[0]
    invalid = lb == IGNORE
    lb0 = jnp.where(invalid, 0, lb)

    x0 = logits_ref[0, 0]
    m = x0
    for c in range(1, C):
        m = jnp.maximum(m, logits_ref[0, c])

    s = jnp.zeros_like(m)
    xl = jnp.zeros_like(m)
    for c in range(C):
        xc = logits_ref[0, c]
        s = s + jnp.exp(xc - m)
        xl = xl + jnp.where(lb0 == c, xc, 0.0)

    pick = jnp.exp(xl - m) / s
    pick = jnp.where(invalid, 1.0, pick)
    nll = m + jnp.log(s) - xl
    key = lax.bitcast_convert_type(pick, jnp.int32)
    key_ref[0] = key
    nll_ref[0] = nll
    acc_ref[0] += jnp.sum((key < _KEY_07).astype(jnp.int32))

    @pl.when(step == pl.num_programs(0) * pl.num_programs(1) - 1)
    def _():
        cnt_ref[0, 0] = acc_ref[0]


def _nll_pick(logits, labels):
    grid = (N, H // _BH)
    return pl.pallas_call(
        _nll_pick_body,
        grid=grid,
        in_specs=[
            pl.BlockSpec((1, C, _BH, W), lambda n, h: (n, 0, h, 0)),
            pl.BlockSpec((1, _BH, W), lambda n, h: (n, h, 0)),
        ],
        out_specs=[
            pl.BlockSpec((1, _BH, W), lambda n, h: (n, h, 0)),
            pl.BlockSpec((1, _BH, W), lambda n, h: (n, h, 0)),
            pl.BlockSpec(memory_space=pltpu.SMEM),
        ],
        out_shape=[
            jax.ShapeDtypeStruct((N, H, W), jnp.int32),
            jax.ShapeDtypeStruct((N, H, W), jnp.float32),
            jax.ShapeDtypeStruct((1, 1), jnp.int32),
        ],
        scratch_shapes=[pltpu.SMEM((1,), jnp.int32)],
    )(logits, labels)


_NB = 2048
_NBP = _NB + 1
_TROWS = H // 2
_CROWS = 64
_NCHUNK = _TROWS // _CROWS


def _lanes():
    return lax.iota(jnp.int32, 16)


def _shr(key, amount):
    return lax.shift_right_logical(key, jnp.full((16,), amount, jnp.int32))


def _zero_hist(hist):
    z = jnp.zeros((16,), jnp.int32)

    @plsc.parallel_loop(0, 16 * (_NB // 16), unroll=8)
    def _(i):
        hist[i >> 7, pl.ds((i & 127) * 16, 16)] = z


def _level(keys_hbm, buf, hist, mbuf, total, shared, sid, img, half, rank,
           bin_fn, mask_fn):
    lanes = _lanes()
    ones = jnp.full((16,), 1, jnp.int32)

    _zero_hist(hist)

    def chunk_body(ch, _):
        pltpu.sync_copy(
            keys_hbm.at[img, pl.ds(half * _TROWS + ch * _CROWS, _CROWS), :],
            buf)

        @plsc.parallel_loop(0, _CROWS * (W // 16), unroll=8)
        def _(i):
            key = buf[i >> 5, pl.ds((i & 31) * 16, 16)]
            plsc.addupdate_scatter(hist, [lanes, bin_fn(key)], ones,
                                   mask=mask_fn(key))

        return 0

    lax.fori_loop(0, _NCHUNK, chunk_body, 0)

    @plsc.parallel_loop(0, _NB // 16, unroll=2)
    def _(i):
        acc = hist[0, pl.ds(i * 16, 16)]
        for l in range(1, 16):
            acc = acc + hist[l, pl.ds(i * 16, 16)]
        total[pl.ds(i * 16, 16)] = acc

    pltpu.sync_copy(total, shared.at[sid])
    plsc.subcore_barrier()
    pltpu.sync_copy(shared, mbuf)
    plsc.subcore_barrier()

    def scan_body(i, carry):
        prefix, bcnt, cbef = carry
        v = mbuf[0, pl.ds(i * 16, 16)]
        for r in range(1, 16):
            v = v + mbuf[r, pl.ds(i * 16, 16)]
        cs = plsc.cumsum(v) + prefix
        le = cs <= rank
        bcnt = bcnt + jnp.sum(jnp.where(le, 1, 0))
        cbef = cbef + jnp.sum(jnp.where(le, v, 0))
        prefix = jnp.max(cs)
        return prefix, bcnt, cbef

    z = jnp.int32(0)
    _, b, cbefore = lax.fori_loop(0, _NB // 16, scan_body, (z, z, z))
    return b, rank - cbefore


def _true_mask(key):
    return jnp.full((16,), True)


def _select_body(keys_hbm, kth_hbm, buf, hist, mbuf, total, kthbuf, shared):
    cid = lax.axis_index("c")
    sid = lax.axis_index("s")
    img = sid >> 1
    half = sid & 1

    @pl.when(cid == 0)
    def _():
        b1, r1 = _level(keys_hbm, buf, hist, mbuf, total, shared, sid, img,
                        half, jnp.int32(N_MIN), lambda key: _shr(key, 21),
                        _true_mask)

        m1 = jnp.broadcast_to(b1, (16,)).astype(jnp.int32)
        b2, r2 = _level(
            keys_hbm, buf, hist, mbuf, total, shared, sid, img, half, r1,
            lambda key: jnp.bitwise_and(_shr(key, 10),
                                        jnp.full((16,), 0x7FF, jnp.int32)),
            lambda key: _shr(key, 21) == m1)

        p2 = b1 * 2048 + b2
        m2 = jnp.broadcast_to(p2, (16,)).astype(jnp.int32)
        b3, _r3 = _level(
            keys_hbm, buf, hist, mbuf, total, shared, sid, img, half, r2,
            lambda key: jnp.bitwise_and(key,
                                        jnp.full((16,), 0x3FF, jnp.int32)),
            lambda key: _shr(key, 10) == m2)

        @pl.when(sid == 0)
        def _():
            kthbuf[...] = jnp.broadcast_to(p2 * 1024 + b3,
                                           (16,)).astype(jnp.int32)
            pltpu.sync_copy(kthbuf, kth_hbm)


@functools.cache
def _sc_select():
    mesh = plsc.VectorSubcoreMesh(core_axis_name="c", subcore_axis_name="s")
    cp = pltpu.CompilerParams(needs_layout_passes=False)
    i32 = jnp.int32
    return pl.kernel(
        _select_body, mesh=mesh, compiler_params=cp,
        out_type=jax.ShapeDtypeStruct((16,), i32),
        scratch_types=[
            pltpu.VMEM((_CROWS, W), i32),
            pltpu.VMEM((16, _NBP), i32),
            pltpu.VMEM((16, _NB), i32),
            pltpu.VMEM((_NB,), i32),
            pltpu.VMEM((16,), i32),
            pltpu.VMEM_SHARED((16, _NB), i32),
        ],
    )


def _loss_body(kth_ref, key_ref, nll_ref, out_ref, acc_ref):
    step = pl.program_id(0)

    @pl.when(step == 0)
    def _():
        acc_ref[0] = 0.0
        acc_ref[1] = 0.0

    thresh_key = jnp.maximum(kth_ref[0, 0], _KEY_07)
    k = key_ref[0]
    nl = nll_ref[0]
    keep = k <= thresh_key
    acc_ref[0] += jnp.sum(jnp.where(keep, nl, 0.0))
    acc_ref[1] += jnp.sum(keep.astype(jnp.float32))

    @pl.when(step == pl.num_programs(0) - 1)
    def _():
        out_ref[0, 0] = acc_ref[0] / jnp.maximum(acc_ref[1], 1.0)


def _masked_ce(kth, keys, nll):
    return pl.pallas_call(
        _loss_body,
        grid=(N,),
        in_specs=[
            pl.BlockSpec(memory_space=pltpu.SMEM),
            pl.BlockSpec((1, H, W), lambda i: (i, 0, 0)),
            pl.BlockSpec((1, H, W), lambda i: (i, 0, 0)),
        ],
        out_specs=pl.BlockSpec(memory_space=pltpu.SMEM),
        out_shape=jax.ShapeDtypeStruct((1, 1), jnp.float32),
        scratch_shapes=[pltpu.SMEM((2,), jnp.float32)],
    )(kth, keys, nll)


def kernel(logits, labels):
    keys, nll, cnt = _nll_pick(logits, labels)
    kth = lax.cond(
        cnt[0, 0] > N_MIN,
        lambda: jnp.full((16,), _KEY_07, jnp.int32),
        lambda: _sc_select()(keys),
    )
    loss = _masked_ce(kth.reshape(1, 16), keys, nll)
    return loss.reshape(())
